# baseline (speedup 1.0000x reference)
import functools

import jax
import jax.numpy as jnp
from jax import lax
from jax.experimental import pallas as pl
from jax.experimental.pallas import tpu as pltpu
from jax.experimental.pallas import tpu_sc as plsc

_NC = 2
_NS = 16
_CHUNK = 128
_RING = 2
_GRP = 16


def _scale_body(f_ref, n_ref, o_ref):
    o_ref[...] = f_ref[...] * n_ref[...]


def _finish_body(p_ref, n_ref, w_ref, b_ref, o_ref):
    acc = (p_ref[0] + p_ref[1]) * n_ref[...]
    z = lax.dot_general(acc, w_ref[...], (((1,), (1,)), ((), ())),
                        preferred_element_type=jnp.float32)
    o_ref[...] = jnp.tanh(z + b_ref[...])


def _sc_segment_sum_body(n_pad, kk,
                         fn_hbm, src_hbm, dst_hbm, zz_hbm, out_hbm,
                         accum_sh, sidx_v, didx_v, *ring):
    bufs = ring[:_RING]
    gsems = ring[_RING:2 * _RING]
    ssems = ring[2 * _RING:]
    c = lax.axis_index("c")
    s = lax.axis_index("s")
    rows_per_tile = n_pad // _NS
    kc = kk // 2
    base = (c * _NS + s) * kc
    ngroups = kc // _GRP
    passes = _GRP // _RING

    pltpu.sync_copy(zz_hbm, accum_sh.at[pl.ds(s * rows_per_tile, rows_per_tile)])
    plsc.subcore_barrier()

    def group(g, carry):
        off = base + g * _GRP
        pltpu.sync_copy(src_hbm.at[pl.ds(off, _GRP)], sidx_v)
        pltpu.sync_copy(dst_hbm.at[pl.ds(off, _GRP)], didx_v)

        for r in range(_RING):
            pltpu.async_copy(fn_hbm.at[sidx_v.at[r]], bufs[r], gsems[r])

        def pass_(q, c2):
            for r in range(_RING):
                j = q * _RING + r
                pltpu.make_async_copy(fn_hbm.at[sidx_v.at[j]],
                                      bufs[r], gsems[r]).wait()
                pltpu.async_copy(bufs[r], accum_sh.at[didx_v.at[j]],
                                 ssems[r], add=True)
            for r in range(_RING):
                j = q * _RING + r
                pltpu.make_async_copy(bufs[r], accum_sh.at[didx_v.at[j]],
                                      ssems[r]).wait()

                @pl.when(q < passes - 1)
                def _(r=r, j=j):
                    pltpu.async_copy(fn_hbm.at[sidx_v.at[j + _RING]],
                                     bufs[r], gsems[r])

            return c2

        lax.fori_loop(0, passes, pass_, 0)
        return carry

    lax.fori_loop(0, ngroups, group, 0)

    plsc.subcore_barrier()
    pltpu.sync_copy(accum_sh.at[pl.ds(s * rows_per_tile, rows_per_tile)],
                    out_hbm.at[c, pl.ds(s * rows_per_tile, rows_per_tile)])


def kernel(features, edge_index, norm, W, b):
    n, d = features.shape
    e = edge_index.shape[1]

    n_pad = -(-(n + 1) // (_NS * 8)) * (_NS * 8)
    kk = -(-e // (_NS * _CHUNK))
    kk = 2 * (-(-kk // (2 * _GRP)) * _GRP)
    e_pad = _NS * kk * _CHUNK

    src = edge_index[0].astype(jnp.int32)
    dst = edge_index[1].astype(jnp.int32)
    pad_i = jnp.arange(e_pad - e, dtype=jnp.int32)
    src_p = jnp.concatenate([src, pad_i % n])
    dst_p = jnp.concatenate([dst, n + pad_i % (n_pad - n)])
    src2 = src_p.reshape(_NS * kk, _CHUNK)
    dst2 = dst_p.reshape(_NS * kk, _CHUNK)
    zz = jnp.zeros((n_pad // _NS, d), jnp.float32)

    rb = 1000
    fn = pl.pallas_call(
        _scale_body,
        grid=(n // rb,),
        in_specs=[pl.BlockSpec((rb, d), lambda i: (i, 0)),
                  pl.BlockSpec((rb, 1), lambda i: (i, 0))],
        out_specs=pl.BlockSpec((rb, d), lambda i: (i, 0)),
        out_shape=jax.ShapeDtypeStruct((n, d), jnp.float32),
    )(features, norm)

    mesh = plsc.VectorSubcoreMesh(core_axis_name="c", subcore_axis_name="s")
    sc_call = pl.kernel(
        functools.partial(_sc_segment_sum_body, n_pad, kk),
        out_type=jax.ShapeDtypeStruct((_NC, n_pad, d), jnp.float32),
        mesh=mesh,
        scratch_types=[
            pltpu.VMEM_SHARED((n_pad, d), jnp.float32),
            pltpu.VMEM((_GRP, _CHUNK), jnp.int32),
            pltpu.VMEM((_GRP, _CHUNK), jnp.int32),
        ] + [pltpu.VMEM((_CHUNK, d), jnp.float32) for _ in range(_RING)]
          + [pltpu.SemaphoreType.DMA for _ in range(2 * _RING)],
    )
    partials = sc_call(fn, src2, dst2, zz)

    b2 = b.reshape(1, d)
    h = pl.pallas_call(
        _finish_body,
        grid=(n // rb,),
        in_specs=[pl.BlockSpec((2, rb, d), lambda i: (0, i, 0)),
                  pl.BlockSpec((rb, 1), lambda i: (i, 0)),
                  pl.BlockSpec((d, d), lambda i: (0, 0)),
                  pl.BlockSpec((1, d), lambda i: (0, 0))],
        out_specs=pl.BlockSpec((rb, d), lambda i: (i, 0)),
        out_shape=jax.ShapeDtypeStruct((n, d), jnp.float32),
    )(partials, norm, W, b2)
    return h


# speedup vs baseline: 1.1291x; 1.1291x over previous
import functools

import jax
import jax.numpy as jnp
from jax import lax
from jax.experimental import pallas as pl
from jax.experimental.pallas import tpu as pltpu
from jax.experimental.pallas import tpu_sc as plsc

_NC = 2
_NS = 16
_CHUNK = 64
_RING = 4
_GRP = 32


def _scale_body(f_ref, n_ref, o_ref):
    o_ref[...] = f_ref[...] * n_ref[...]


def _finish_body(p_ref, n_ref, w_ref, b_ref, o_ref):
    acc = (p_ref[0] + p_ref[1]) * n_ref[...]
    z = lax.dot_general(acc, w_ref[...], (((1,), (1,)), ((), ())),
                        preferred_element_type=jnp.float32)
    o_ref[...] = jnp.tanh(z + b_ref[...])


def _sc_segment_sum_body(n_pad, kk,
                         fn_hbm, src_hbm, dst_hbm, zz_hbm, out_hbm,
                         accum_sh, sidx_v, didx_v, *ring):
    bufs = ring[:_RING]
    gsems = ring[_RING:2 * _RING]
    ssems = ring[2 * _RING:]
    c = lax.axis_index("c")
    s = lax.axis_index("s")
    rows_per_tile = n_pad // _NS
    kc = kk // 2
    base = (c * _NS + s) * kc
    ngroups = kc // _GRP
    passes = _GRP // _RING

    pltpu.sync_copy(zz_hbm, accum_sh.at[pl.ds(s * rows_per_tile, rows_per_tile)])
    plsc.subcore_barrier()

    def group(g, carry):
        off = base + g * _GRP
        pltpu.sync_copy(src_hbm.at[pl.ds(off, _GRP)], sidx_v)
        pltpu.sync_copy(dst_hbm.at[pl.ds(off, _GRP)], didx_v)

        for r in range(_RING):
            pltpu.async_copy(fn_hbm.at[sidx_v.at[r]], bufs[r], gsems[r])

        def pass_(q, c2):
            for r in range(_RING):
                j = q * _RING + r
                pltpu.make_async_copy(fn_hbm.at[sidx_v.at[j]],
                                      bufs[r], gsems[r]).wait()
                pltpu.async_copy(bufs[r], accum_sh.at[didx_v.at[j]],
                                 ssems[r], add=True)
            for r in range(_RING):
                j = q * _RING + r
                pltpu.make_async_copy(bufs[r], accum_sh.at[didx_v.at[j]],
                                      ssems[r]).wait()

                @pl.when(q < passes - 1)
                def _(r=r, j=j):
                    pltpu.async_copy(fn_hbm.at[sidx_v.at[j + _RING]],
                                     bufs[r], gsems[r])

            return c2

        lax.fori_loop(0, passes, pass_, 0)
        return carry

    lax.fori_loop(0, ngroups, group, 0)

    plsc.subcore_barrier()
    pltpu.sync_copy(accum_sh.at[pl.ds(s * rows_per_tile, rows_per_tile)],
                    out_hbm.at[c, pl.ds(s * rows_per_tile, rows_per_tile)])


def kernel(features, edge_index, norm, W, b):
    n, d = features.shape
    e = edge_index.shape[1]

    n_pad = -(-(n + 1) // (_NS * 8)) * (_NS * 8)
    kk = -(-e // (_NS * _CHUNK))
    kk = 2 * (-(-kk // (2 * _GRP)) * _GRP)
    e_pad = _NS * kk * _CHUNK

    src = edge_index[0].astype(jnp.int32)
    dst = edge_index[1].astype(jnp.int32)
    pad_i = jnp.arange(e_pad - e, dtype=jnp.int32)
    src_p = jnp.concatenate([src, pad_i % n])
    dst_p = jnp.concatenate([dst, n + pad_i % (n_pad - n)])
    src2 = src_p.reshape(_NS * kk, _CHUNK)
    dst2 = dst_p.reshape(_NS * kk, _CHUNK)
    zz = jnp.zeros((n_pad // _NS, d), jnp.float32)

    rb = 1000
    fn = pl.pallas_call(
        _scale_body,
        grid=(n // rb,),
        in_specs=[pl.BlockSpec((rb, d), lambda i: (i, 0)),
                  pl.BlockSpec((rb, 1), lambda i: (i, 0))],
        out_specs=pl.BlockSpec((rb, d), lambda i: (i, 0)),
        out_shape=jax.ShapeDtypeStruct((n, d), jnp.float32),
    )(features, norm)

    mesh = plsc.VectorSubcoreMesh(core_axis_name="c", subcore_axis_name="s")
    sc_call = pl.kernel(
        functools.partial(_sc_segment_sum_body, n_pad, kk),
        out_type=jax.ShapeDtypeStruct((_NC, n_pad, d), jnp.float32),
        mesh=mesh,
        scratch_types=[
            pltpu.VMEM_SHARED((n_pad, d), jnp.float32),
            pltpu.VMEM((_GRP, _CHUNK), jnp.int32),
            pltpu.VMEM((_GRP, _CHUNK), jnp.int32),
        ] + [pltpu.VMEM((_CHUNK, d), jnp.float32) for _ in range(_RING)]
          + [pltpu.SemaphoreType.DMA for _ in range(2 * _RING)],
    )
    partials = sc_call(fn, src2, dst2, zz)

    b2 = b.reshape(1, d)
    h = pl.pallas_call(
        _finish_body,
        grid=(n // rb,),
        in_specs=[pl.BlockSpec((2, rb, d), lambda i: (0, i, 0)),
                  pl.BlockSpec((rb, 1), lambda i: (i, 0)),
                  pl.BlockSpec((d, d), lambda i: (0, 0)),
                  pl.BlockSpec((1, d), lambda i: (0, 0))],
        out_specs=pl.BlockSpec((rb, d), lambda i: (i, 0)),
        out_shape=jax.ShapeDtypeStruct((n, d), jnp.float32),
    )(partials, norm, W, b2)
    return h


# speedup vs baseline: 1.1419x; 1.0114x over previous
import functools

import jax
import jax.numpy as jnp
from jax import lax
from jax.experimental import pallas as pl
from jax.experimental.pallas import tpu as pltpu
from jax.experimental.pallas import tpu_sc as plsc

_NC = 2
_NS = 16
_CHUNK = 64
_RING = 4
_GRP = 40


def _scale_body(f_ref, n_ref, o_ref):
    o_ref[...] = f_ref[...] * n_ref[...]


def _finish_body(p_ref, n_ref, w_ref, b_ref, o_ref):
    acc = (p_ref[0] + p_ref[1]) * n_ref[...]
    z = lax.dot_general(acc, w_ref[...], (((1,), (1,)), ((), ())),
                        preferred_element_type=jnp.float32)
    o_ref[...] = jnp.tanh(z + b_ref[...])


def _sc_segment_sum_body(n_pad, kk,
                         fn_hbm, src_hbm, dst_hbm, zz_hbm, out_hbm,
                         accum_sh, sidx_v, didx_v, *ring):
    bufs = ring[:_RING]
    gsems = ring[_RING:2 * _RING]
    ssems = ring[2 * _RING:]
    c = lax.axis_index("c")
    s = lax.axis_index("s")
    rows_per_tile = n_pad // _NS
    kc = kk // 2
    base = (c * _NS + s) * kc
    ngroups = kc // _GRP
    passes = _GRP // _RING

    pltpu.sync_copy(zz_hbm, accum_sh.at[pl.ds(s * rows_per_tile, rows_per_tile)])
    plsc.subcore_barrier()

    def group(g, carry):
        off = base + g * _GRP
        pltpu.sync_copy(src_hbm.at[pl.ds(off, _GRP)], sidx_v)
        pltpu.sync_copy(dst_hbm.at[pl.ds(off, _GRP)], didx_v)

        for r in range(_RING):
            pltpu.async_copy(fn_hbm.at[sidx_v.at[r]], bufs[r], gsems[r])

        def pass_(q, c2):
            for r in range(_RING):
                j = q * _RING + r
                pltpu.make_async_copy(fn_hbm.at[sidx_v.at[j]],
                                      bufs[r], gsems[r]).wait()
                pltpu.async_copy(bufs[r], accum_sh.at[didx_v.at[j]],
                                 ssems[r], add=True)
            for r in range(_RING):
                j = q * _RING + r
                pltpu.make_async_copy(bufs[r], accum_sh.at[didx_v.at[j]],
                                      ssems[r]).wait()

                @pl.when(q < passes - 1)
                def _(r=r, j=j):
                    pltpu.async_copy(fn_hbm.at[sidx_v.at[j + _RING]],
                                     bufs[r], gsems[r])

            return c2

        lax.fori_loop(0, passes, pass_, 0)
        return carry

    lax.fori_loop(0, ngroups, group, 0)

    plsc.subcore_barrier()
    pltpu.sync_copy(accum_sh.at[pl.ds(s * rows_per_tile, rows_per_tile)],
                    out_hbm.at[c, pl.ds(s * rows_per_tile, rows_per_tile)])


def kernel(features, edge_index, norm, W, b):
    n, d = features.shape
    e = edge_index.shape[1]

    n_pad = -(-(n + 1) // (_NS * 8)) * (_NS * 8)
    kk = -(-e // (_NS * _CHUNK))
    kk = 2 * (-(-kk // (2 * _GRP)) * _GRP)
    e_pad = _NS * kk * _CHUNK

    src = edge_index[0].astype(jnp.int32)
    dst = edge_index[1].astype(jnp.int32)
    pad_i = jnp.arange(e_pad - e, dtype=jnp.int32)
    src_p = jnp.concatenate([src, pad_i % n])
    dst_p = jnp.concatenate([dst, n + pad_i % (n_pad - n)])
    src2 = src_p.reshape(_NS * kk, _CHUNK)
    dst2 = dst_p.reshape(_NS * kk, _CHUNK)
    zz = jnp.zeros((n_pad // _NS, d), jnp.float32)

    rb = 1000
    fn = pl.pallas_call(
        _scale_body,
        grid=(n // rb,),
        in_specs=[pl.BlockSpec((rb, d), lambda i: (i, 0)),
                  pl.BlockSpec((rb, 1), lambda i: (i, 0))],
        out_specs=pl.BlockSpec((rb, d), lambda i: (i, 0)),
        out_shape=jax.ShapeDtypeStruct((n, d), jnp.float32),
    )(features, norm)

    mesh = plsc.VectorSubcoreMesh(core_axis_name="c", subcore_axis_name="s")
    sc_call = pl.kernel(
        functools.partial(_sc_segment_sum_body, n_pad, kk),
        out_type=jax.ShapeDtypeStruct((_NC, n_pad, d), jnp.float32),
        mesh=mesh,
        scratch_types=[
            pltpu.VMEM_SHARED((n_pad, d), jnp.float32),
            pltpu.VMEM((_GRP, _CHUNK), jnp.int32),
            pltpu.VMEM((_GRP, _CHUNK), jnp.int32),
        ] + [pltpu.VMEM((_CHUNK, d), jnp.float32) for _ in range(_RING)]
          + [pltpu.SemaphoreType.DMA for _ in range(2 * _RING)],
    )
    partials = sc_call(fn, src2, dst2, zz)

    b2 = b.reshape(1, d)
    h = pl.pallas_call(
        _finish_body,
        grid=(n // rb,),
        in_specs=[pl.BlockSpec((2, rb, d), lambda i: (0, i, 0)),
                  pl.BlockSpec((rb, 1), lambda i: (i, 0)),
                  pl.BlockSpec((d, d), lambda i: (0, 0)),
                  pl.BlockSpec((1, d), lambda i: (0, 0))],
        out_specs=pl.BlockSpec((rb, d), lambda i: (i, 0)),
        out_shape=jax.ShapeDtypeStruct((n, d), jnp.float32),
    )(partials, norm, W, b2)
    return h
